# baseline (speedup 1.0000x reference)
import functools

import jax
import jax.numpy as jnp
from jax import lax
from jax.experimental import pallas as pl
from jax.experimental.pallas import tpu as pltpu
from jax.experimental.pallas import tpu_sc as plsc

_NLANE = 16


def _sc_body(nodes_hbm, mask_hbm, ker_hbm, enc_hbm, out_hbm,
             nodes_v, mask_v, ker_v, enc_v, histT, hist, offs, pcur,
             senc, snode, co, lens, invl, W, L, NC):
    N = W * L
    V = 1024
    NCH = N // _NLANE
    CELLS = W * W

    wid = lax.axis_index("s") * NC + lax.axis_index("c")
    pltpu.sync_copy(nodes_hbm.at[wid], nodes_v)
    pltpu.sync_copy(mask_hbm.at[wid], mask_v)
    pltpu.sync_copy(ker_hbm, ker_v)
    pltpu.sync_copy(enc_hbm, enc_v)

    lane = lax.iota(jnp.int32, _NLANE)
    zi = jnp.zeros((_NLANE,), jnp.int32)
    zf = jnp.zeros((_NLANE,), jnp.float32)

    def zero_body(i, _):
        for j in range(8):
            co[pl.ds(i * 128 + j * _NLANE, _NLANE)] = zf
            histT[pl.ds(i * 128 + j * _NLANE, _NLANE)] = zi
        return 0
    lax.fori_loop(0, CELLS // 128, zero_body, 0)

    def zero_small(i, _):
        lens[pl.ds(i * _NLANE, _NLANE)] = zf
        return 0
    lax.fori_loop(0, W // _NLANE, zero_small, 0)

    ones_i = jnp.ones((_NLANE,), jnp.int32)

    def hist_body(i, _):
        for j in range(4):
            b = i * 4 * _NLANE + j * _NLANE
            idx = nodes_v[pl.ds(b, _NLANE)]
            mval = mask_v[pl.ds(b, _NLANE)]
            valid = mval != 0.0
            w = enc_v[pl.ds(b, _NLANE)] >> 8
            plsc.addupdate_scatter(lens, [w], mval)
            plsc.addupdate_scatter(histT, [lane * V + idx], ones_i, mask=valid)
        return 0
    lax.fori_loop(0, NCH // 4, hist_body, 0)

    def seg_body(j, carry):
        b = j * _NLANE
        acc = histT[pl.ds(b, _NLANE)]
        for l in range(1, _NLANE):
            acc = acc + histT[pl.ds(l * V + b, _NLANE)]
        hist[pl.ds(b, _NLANE)] = acc
        cs = plsc.cumsum(acc)
        off = cs - acc + carry
        offs[pl.ds(b, _NLANE)] = off
        for l in range(_NLANE):
            pcur[pl.ds(l * V + b, _NLANE)] = off
            off = off + histT[pl.ds(l * V + b, _NLANE)]
        return carry + jnp.sum(acc)
    nvalid = lax.fori_loop(0, V // _NLANE, seg_body, jnp.int32(0))

    def place_body(i, _):
        for j in range(4):
            b = i * 4 * _NLANE + j * _NLANE
            idx = nodes_v[pl.ds(b, _NLANE)]
            valid = mask_v[pl.ds(b, _NLANE)] != 0.0
            ev = enc_v[pl.ds(b, _NLANE)]
            flat = lane * V + idx
            pos = plsc.load_gather(pcur, [flat], mask=valid)
            plsc.store_scatter(senc, [pos], ev, mask=valid)
            plsc.store_scatter(snode, [pos], idx, mask=valid)
            plsc.store_scatter(pcur, [flat], pos + 1, mask=valid)
        return 0
    lax.fori_loop(0, NCH // 4, place_body, 0)

    def invl_body(i, _):
        b = i * _NLANE
        lv = lens[pl.ds(b, _NLANE)]
        invl[pl.ds(b, _NLANE)] = 1.0 / jnp.maximum(lv, 1.0)
        return 0
    lax.fori_loop(0, W // _NLANE, invl_body, 0)

    def emit_body(i, _):
        b = i * _NLANE
        svec = b + lane
        act0 = svec < nvalid
        encs = senc[pl.ds(b, _NLANE)]
        vs = snode[pl.ds(b, _NLANE)]
        a = plsc.load_gather(offs, [vs], mask=act0)
        m = plsc.load_gather(hist, [vs], mask=act0)
        m = jnp.where(act0, m, 0)
        mz = jnp.where(m >= 2, m, 0)
        mlen = jnp.max(mz)
        cell0 = (encs >> 8) * W
        p20 = (encs & 255) * L

        def cond(c):
            return c[0] < mlen

        def body(c):
            k = c[0]
            for u in range(4):
                ku = k + u
                act = ku < mz
                je = plsc.load_gather(senc, [a + ku], mask=act)
                kv = plsc.load_gather(ker_v, [p20 + (je & 255)], mask=act)
                plsc.addupdate_scatter(co, [cell0 + (je >> 8)], kv, mask=act)
            return (k + 4,)
        lax.while_loop(cond, body, (jnp.int32(0),))
        return 0
    lax.fori_loop(0, NCH, emit_body, 0)

    def ep_body(i, _):
        w1 = jnp.broadcast_to(i.astype(jnp.int32), (_NLANE,))
        il1 = plsc.load_gather(invl, [w1])
        for j in range(W // _NLANE):
            c0 = i * W + j * _NLANE
            x = co[pl.ds(c0, _NLANE)]
            il2 = invl[pl.ds(j * _NLANE, _NLANE)]
            y = x * il1 * il2
            y = jnp.clip(y, -10.0, 10.0)
            e = jnp.exp(2.0 * y)
            co[pl.ds(c0, _NLANE)] = 1.0 - 2.0 / (e + 1.0)
        return 0
    lax.fori_loop(0, W, ep_body, 0)
    pltpu.sync_copy(co, out_hbm.at[wid])


def kernel(anonymized_nodes, walk_masks, kernel):
    B, W, L = anonymized_nodes.shape
    N = W * L
    V = 1024
    info = plsc.get_sparse_core_info()
    NC = info.num_cores
    ker = jnp.clip(kernel[:L, :L], -10.0, 10.0).reshape(L * L)
    nodes_flat = anonymized_nodes.reshape(B, N)
    mask_flat = walk_masks.reshape(B, N)
    ar = jnp.arange(N, dtype=jnp.int32)
    enc = (ar // L) * 256 + (ar % L)

    mesh = plsc.VectorSubcoreMesh(core_axis_name="c", subcore_axis_name="s")
    body = functools.partial(_sc_body, W=W, L=L, NC=NC)
    run = pl.kernel(
        body,
        out_type=jax.ShapeDtypeStruct((B, W * W), jnp.float32),
        mesh=mesh,
        scratch_types=[
            pltpu.VMEM((N,), jnp.int32),
            pltpu.VMEM((N,), jnp.float32),
            pltpu.VMEM((L * L,), jnp.float32),
            pltpu.VMEM((N,), jnp.int32),
            pltpu.VMEM((_NLANE * V,), jnp.int32),
            pltpu.VMEM((V,), jnp.int32),
            pltpu.VMEM((V,), jnp.int32),
            pltpu.VMEM((_NLANE * V,), jnp.int32),
            pltpu.VMEM((N,), jnp.int32),
            pltpu.VMEM((N,), jnp.int32),
            pltpu.VMEM((W * W,), jnp.float32),
            pltpu.VMEM((W,), jnp.float32),
            pltpu.VMEM((W,), jnp.float32),
        ],
        compiler_params=pltpu.CompilerParams(needs_layout_passes=False),
    )
    out = run(nodes_flat, mask_flat, ker, enc)
    return out.reshape(B, W, W)


# speedup vs baseline: 1.0179x; 1.0179x over previous
import functools

import jax
import jax.numpy as jnp
from jax import lax
from jax.experimental import pallas as pl
from jax.experimental.pallas import tpu as pltpu
from jax.experimental.pallas import tpu_sc as plsc

_NLANE = 16


def _sc_body(nodes_hbm, mask_hbm, ker_hbm, enc_hbm, out_hbm,
             nodes_v, mask_v, ker_v, enc_v, histT, hist, offs, pcur,
             senc, snode, co, lens, invl, W, L, NC):
    N = W * L
    V = 1024
    NCH = N // _NLANE
    CELLS = W * W

    wid = lax.axis_index("s") * NC + lax.axis_index("c")
    pltpu.sync_copy(nodes_hbm.at[wid], nodes_v)
    pltpu.sync_copy(mask_hbm.at[wid], mask_v)
    pltpu.sync_copy(ker_hbm, ker_v)
    pltpu.sync_copy(enc_hbm, enc_v)

    lane = lax.iota(jnp.int32, _NLANE)
    zi = jnp.zeros((_NLANE,), jnp.int32)
    zf = jnp.zeros((_NLANE,), jnp.float32)

    def zero_body(i, _):
        for j in range(8):
            co[pl.ds(i * 128 + j * _NLANE, _NLANE)] = zf
            histT[pl.ds(i * 128 + j * _NLANE, _NLANE)] = zi
        return 0
    lax.fori_loop(0, CELLS // 128, zero_body, 0)

    def zero_small(i, _):
        lens[pl.ds(i * _NLANE, _NLANE)] = zf
        return 0
    lax.fori_loop(0, W // _NLANE, zero_small, 0)

    ones_i = jnp.ones((_NLANE,), jnp.int32)

    def hist_body(i, _):
        for j in range(4):
            b = i * 4 * _NLANE + j * _NLANE
            idx = nodes_v[pl.ds(b, _NLANE)]
            mval = mask_v[pl.ds(b, _NLANE)]
            valid = mval != 0.0
            w = enc_v[pl.ds(b, _NLANE)] >> 8
            plsc.addupdate_scatter(lens, [w], mval)
            plsc.addupdate_scatter(histT, [lane * V + idx], ones_i, mask=valid)
        return 0
    lax.fori_loop(0, NCH // 4, hist_body, 0)

    def seg_body(j, carry):
        b = j * _NLANE
        acc = histT[pl.ds(b, _NLANE)]
        for l in range(1, _NLANE):
            acc = acc + histT[pl.ds(l * V + b, _NLANE)]
        hist[pl.ds(b, _NLANE)] = acc
        cs = plsc.cumsum(acc)
        off = cs - acc + carry
        offs[pl.ds(b, _NLANE)] = off
        for l in range(_NLANE):
            pcur[pl.ds(l * V + b, _NLANE)] = off
            off = off + histT[pl.ds(l * V + b, _NLANE)]
        return carry + jnp.sum(acc)
    nvalid = lax.fori_loop(0, V // _NLANE, seg_body, jnp.int32(0))

    def place_body(i, _):
        for j in range(4):
            b = i * 4 * _NLANE + j * _NLANE
            idx = nodes_v[pl.ds(b, _NLANE)]
            valid = mask_v[pl.ds(b, _NLANE)] != 0.0
            ev = enc_v[pl.ds(b, _NLANE)]
            flat = lane * V + idx
            pos = plsc.load_gather(pcur, [flat], mask=valid)
            plsc.store_scatter(senc, [pos], ev, mask=valid)
            plsc.store_scatter(snode, [pos], idx, mask=valid)
            plsc.store_scatter(pcur, [flat], pos + 1, mask=valid)
        return 0
    lax.fori_loop(0, NCH // 4, place_body, 0)

    def invl_body(i, _):
        b = i * _NLANE
        lv = lens[pl.ds(b, _NLANE)]
        invl[pl.ds(b, _NLANE)] = 1.0 / jnp.maximum(lv, 1.0)
        return 0
    lax.fori_loop(0, W // _NLANE, invl_body, 0)

    def emit_body(i, _):
        b = i * _NLANE
        svec = b + lane
        act0 = svec < nvalid
        encs = senc[pl.ds(b, _NLANE)]
        vs = snode[pl.ds(b, _NLANE)]
        a = plsc.load_gather(offs, [vs], mask=act0)
        m = plsc.load_gather(hist, [vs], mask=act0)
        m = jnp.where(act0, m, 0)
        mz = jnp.where(m >= 2, m, 0)
        mlen = jnp.max(mz)
        cell0 = (encs >> 8) * W
        p20 = (encs & 255) * L

        def cond(c):
            return c[0] < mlen

        def body(c):
            k = c[0]
            for u in range(2):
                ku = k + u
                act = ku < mz
                je = plsc.load_gather(senc, [a + ku], mask=act)
                kv = plsc.load_gather(ker_v, [p20 + (je & 255)], mask=act)
                plsc.addupdate_scatter(co, [cell0 + (je >> 8)], kv, mask=act)
            return (k + 2,)
        lax.while_loop(cond, body, (jnp.int32(0),))
        return 0
    lax.fori_loop(0, NCH, emit_body, 0)

    def ep_body(i, _):
        w1 = jnp.broadcast_to(i.astype(jnp.int32), (_NLANE,))
        il1 = plsc.load_gather(invl, [w1])
        for j in range(W // _NLANE):
            c0 = i * W + j * _NLANE
            x = co[pl.ds(c0, _NLANE)]
            il2 = invl[pl.ds(j * _NLANE, _NLANE)]
            y = x * il1 * il2
            y = jnp.clip(y, -10.0, 10.0)
            e = jnp.exp(2.0 * y)
            co[pl.ds(c0, _NLANE)] = 1.0 - 2.0 / (e + 1.0)
        return 0
    lax.fori_loop(0, W, ep_body, 0)
    pltpu.sync_copy(co, out_hbm.at[wid])


def kernel(anonymized_nodes, walk_masks, kernel):
    B, W, L = anonymized_nodes.shape
    N = W * L
    V = 1024
    info = plsc.get_sparse_core_info()
    NC = info.num_cores
    ker = jnp.clip(kernel[:L, :L], -10.0, 10.0).reshape(L * L)
    nodes_flat = anonymized_nodes.reshape(B, N)
    mask_flat = walk_masks.reshape(B, N)
    ar = jnp.arange(N, dtype=jnp.int32)
    enc = (ar // L) * 256 + (ar % L)

    mesh = plsc.VectorSubcoreMesh(core_axis_name="c", subcore_axis_name="s")
    body = functools.partial(_sc_body, W=W, L=L, NC=NC)
    run = pl.kernel(
        body,
        out_type=jax.ShapeDtypeStruct((B, W * W), jnp.float32),
        mesh=mesh,
        scratch_types=[
            pltpu.VMEM((N,), jnp.int32),
            pltpu.VMEM((N,), jnp.float32),
            pltpu.VMEM((L * L,), jnp.float32),
            pltpu.VMEM((N,), jnp.int32),
            pltpu.VMEM((_NLANE * V,), jnp.int32),
            pltpu.VMEM((V,), jnp.int32),
            pltpu.VMEM((V,), jnp.int32),
            pltpu.VMEM((_NLANE * V,), jnp.int32),
            pltpu.VMEM((N,), jnp.int32),
            pltpu.VMEM((N,), jnp.int32),
            pltpu.VMEM((W * W,), jnp.float32),
            pltpu.VMEM((W,), jnp.float32),
            pltpu.VMEM((W,), jnp.float32),
        ],
        compiler_params=pltpu.CompilerParams(needs_layout_passes=False),
    )
    out = run(nodes_flat, mask_flat, ker, enc)
    return out.reshape(B, W, W)


# speedup vs baseline: 1.0282x; 1.0101x over previous
import functools

import jax
import jax.numpy as jnp
from jax import lax
from jax.experimental import pallas as pl
from jax.experimental.pallas import tpu as pltpu
from jax.experimental.pallas import tpu_sc as plsc

_NLANE = 16


def _sc_body(nodes_hbm, mask_hbm, ker_hbm, enc_hbm, out_hbm,
             nodes_v, mask_v, ker_v, enc_v, histT, hist, offs, pcur,
             senc, snode, co, lens, invl, W, L, NC):
    N = W * L
    V = 1024
    NCH = N // _NLANE
    CELLS = W * W

    wid = lax.axis_index("s") * NC + lax.axis_index("c")
    pltpu.sync_copy(nodes_hbm.at[wid], nodes_v)
    pltpu.sync_copy(mask_hbm.at[wid], mask_v)
    pltpu.sync_copy(ker_hbm, ker_v)
    pltpu.sync_copy(enc_hbm, enc_v)

    lane = lax.iota(jnp.int32, _NLANE)
    zi = jnp.zeros((_NLANE,), jnp.int32)
    zf = jnp.zeros((_NLANE,), jnp.float32)

    def zero_body(i, _):
        for j in range(8):
            co[pl.ds(i * 128 + j * _NLANE, _NLANE)] = zf
            histT[pl.ds(i * 128 + j * _NLANE, _NLANE)] = zi
        return 0
    lax.fori_loop(0, CELLS // 128, zero_body, 0)

    def zero_small(i, _):
        lens[pl.ds(i * _NLANE, _NLANE)] = zf
        return 0
    lax.fori_loop(0, W // _NLANE, zero_small, 0)

    ones_i = jnp.ones((_NLANE,), jnp.int32)

    def hist_body(i, _):
        for j in range(4):
            b = i * 4 * _NLANE + j * _NLANE
            idx = nodes_v[pl.ds(b, _NLANE)]
            mval = mask_v[pl.ds(b, _NLANE)]
            valid = mval != 0.0
            w = enc_v[pl.ds(b, _NLANE)] >> 8
            plsc.addupdate_scatter(lens, [w], mval)
            plsc.addupdate_scatter(histT, [lane * V + idx], ones_i, mask=valid)
        return 0
    lax.fori_loop(0, NCH // 4, hist_body, 0)

    def seg_body(j, carry):
        b = j * _NLANE
        acc = histT[pl.ds(b, _NLANE)]
        for l in range(1, _NLANE):
            acc = acc + histT[pl.ds(l * V + b, _NLANE)]
        hist[pl.ds(b, _NLANE)] = acc
        cs = plsc.cumsum(acc)
        off = cs - acc + carry
        offs[pl.ds(b, _NLANE)] = off
        for l in range(_NLANE):
            pcur[pl.ds(l * V + b, _NLANE)] = off
            off = off + histT[pl.ds(l * V + b, _NLANE)]
        return carry + jnp.sum(acc)
    nvalid = lax.fori_loop(0, V // _NLANE, seg_body, jnp.int32(0))

    def place_body(i, _):
        for j in range(4):
            b = i * 4 * _NLANE + j * _NLANE
            idx = nodes_v[pl.ds(b, _NLANE)]
            valid = mask_v[pl.ds(b, _NLANE)] != 0.0
            ev = enc_v[pl.ds(b, _NLANE)]
            flat = lane * V + idx
            pos = plsc.load_gather(pcur, [flat], mask=valid)
            plsc.store_scatter(senc, [pos], ev, mask=valid)
            plsc.store_scatter(snode, [pos], idx, mask=valid)
            plsc.store_scatter(pcur, [flat], pos + 1, mask=valid)
        return 0
    lax.fori_loop(0, NCH // 4, place_body, 0)

    def invl_body(i, _):
        b = i * _NLANE
        lv = lens[pl.ds(b, _NLANE)]
        invl[pl.ds(b, _NLANE)] = 1.0 / jnp.maximum(lv, 1.0)
        return 0
    lax.fori_loop(0, W // _NLANE, invl_body, 0)

    def emit_body(i, _):
        b = i * _NLANE
        svec = b + lane
        act0 = svec < nvalid
        encs = senc[pl.ds(b, _NLANE)]
        vs = snode[pl.ds(b, _NLANE)]
        a = plsc.load_gather(offs, [vs], mask=act0)
        m = plsc.load_gather(hist, [vs], mask=act0)
        m = jnp.where(act0, m, 0)
        seg_ok = act0 & (m >= 2)
        mlen = jnp.max(m)
        cell0 = (encs >> 8) * W
        p20 = (encs & 255) * L

        def cond(c):
            return c[0] < mlen

        def body(c):
            k = c[0]
            for u in range(2):
                ku = k + u
                act = seg_ok & (ku < m)
                je = plsc.load_gather(senc, [a + ku], mask=act)
                kv = plsc.load_gather(ker_v, [p20 + (je & 255)], mask=act)
                plsc.addupdate_scatter(co, [cell0 + (je >> 8)], kv, mask=act)
            return (k + 2,)
        lax.while_loop(cond, body, (jnp.int32(0),))
        return 0
    lax.fori_loop(0, NCH, emit_body, 0)

    def ep_body(i, _):
        w1 = jnp.broadcast_to(i.astype(jnp.int32), (_NLANE,))
        il1 = plsc.load_gather(invl, [w1])
        for j in range(W // _NLANE):
            c0 = i * W + j * _NLANE
            x = co[pl.ds(c0, _NLANE)]
            il2 = invl[pl.ds(j * _NLANE, _NLANE)]
            y = x * il1 * il2
            y = jnp.clip(y, -10.0, 10.0)
            e = jnp.exp(2.0 * y)
            co[pl.ds(c0, _NLANE)] = 1.0 - 2.0 / (e + 1.0)
        return 0
    lax.fori_loop(0, W, ep_body, 0)
    pltpu.sync_copy(co, out_hbm.at[wid])


def kernel(anonymized_nodes, walk_masks, kernel):
    B, W, L = anonymized_nodes.shape
    N = W * L
    V = 1024
    info = plsc.get_sparse_core_info()
    NC = info.num_cores
    ker = jnp.clip(kernel[:L, :L], -10.0, 10.0).reshape(L * L)
    nodes_flat = anonymized_nodes.reshape(B, N)
    mask_flat = walk_masks.reshape(B, N)
    ar = jnp.arange(N, dtype=jnp.int32)
    enc = (ar // L) * 256 + (ar % L)

    mesh = plsc.VectorSubcoreMesh(core_axis_name="c", subcore_axis_name="s")
    body = functools.partial(_sc_body, W=W, L=L, NC=NC)
    run = pl.kernel(
        body,
        out_type=jax.ShapeDtypeStruct((B, W * W), jnp.float32),
        mesh=mesh,
        scratch_types=[
            pltpu.VMEM((N,), jnp.int32),
            pltpu.VMEM((N,), jnp.float32),
            pltpu.VMEM((L * L,), jnp.float32),
            pltpu.VMEM((N,), jnp.int32),
            pltpu.VMEM((_NLANE * V,), jnp.int32),
            pltpu.VMEM((V,), jnp.int32),
            pltpu.VMEM((V,), jnp.int32),
            pltpu.VMEM((_NLANE * V,), jnp.int32),
            pltpu.VMEM((N,), jnp.int32),
            pltpu.VMEM((N,), jnp.int32),
            pltpu.VMEM((W * W,), jnp.float32),
            pltpu.VMEM((W,), jnp.float32),
            pltpu.VMEM((W,), jnp.float32),
        ],
        compiler_params=pltpu.CompilerParams(needs_layout_passes=False),
    )
    out = run(nodes_flat, mask_flat, ker, enc)
    return out.reshape(B, W, W)
